# baseline (speedup 1.0000x reference)
import jax
import jax.numpy as jnp
from jax.experimental import pallas as pl
from jax.experimental.pallas import tpu as pltpu

B = 1024
D = 128
T = 4
K = 512
M = 65536
TK = T * K

_BB = 1024
_STEPS = B // _BB


def _body(flat_sref, task_ref, feat_ref, cent_ref, bank_ref,
          out_ref, written, sem):
    del bank_ref
    i = pl.program_id(0)
    feats = feat_ref[...]
    cents = cent_ref[...]
    feats_bf = feats.astype(jnp.bfloat16)
    cents_bf = cents.astype(jnp.bfloat16)
    a = jax.lax.dot_general(
        feats_bf, cents_bf * jnp.bfloat16(-2.0), (((1,), (1,)), ((), ())),
        preferred_element_type=jnp.float32)
    sq_col = jax.lax.dot_general(
        cents * cents, jnp.ones((D, 1), jnp.float32), (((1,), (0,)), ((), ())),
        preferred_element_type=jnp.float32)
    task = task_ref[...]
    ut = jax.lax.broadcasted_iota(jnp.int32, (_BB, 5), 1)
    u = jnp.where(ut == 0, 1.0,
                  jnp.where(ut - 1 == task, 0.0, 1.0)).astype(jnp.float32)
    wt = jax.lax.broadcasted_iota(jnp.int32, (TK, 5), 1)
    wc = jax.lax.broadcasted_iota(jnp.int32, (TK, 5), 0)
    big = jnp.float32(1e9)
    w = jnp.where(wt == 0, sq_col,
                  jnp.where(wc // K == wt - 1, big, 0.0)).astype(jnp.float32)
    b2 = jax.lax.dot_general(
        u, w, (((1,), (1,)), ((), ())),
        preferred_element_type=jnp.float32)
    masked = a + b2
    mins = jnp.min(masked, axis=1, keepdims=True)
    onehot = (masked == mins).astype(jnp.bfloat16)
    sel = jax.lax.dot_general(
        onehot, cents_bf, (((1,), (0,)), ((), ())),
        preferred_element_type=jnp.float32)
    written[pl.ds(i * _BB, _BB), :] = feats + 0.1 * sel

    def issue(j, _):
        s = i * _BB + j
        pltpu.make_async_copy(
            written.at[pl.ds(s, 1), :],
            out_ref.at[pl.ds(flat_sref[s], 1), :],
            sem,
        ).start()
        return 0

    jax.lax.fori_loop(0, _BB, issue, 0, unroll=16)

    @pl.when(i == _STEPS - 1)
    def _drain():
        pltpu.make_async_copy(
            written.at[...],
            out_ref.at[pl.ds(0, B), :],
            sem,
        ).wait()


def kernel(features, feature_bank, cluster_centers, task_idx, write_idx):
    flat_centers = cluster_centers.reshape(TK, D)
    task2d = task_idx.reshape(B, 1)
    bank_flat = feature_bank.reshape(T * M, D)
    flat_idx = task_idx * M + write_idx

    grid_spec = pltpu.PrefetchScalarGridSpec(
        num_scalar_prefetch=1,
        grid=(_STEPS,),
        in_specs=[
            pl.BlockSpec((_BB, 1), lambda i, f: (i, 0)),
            pl.BlockSpec((_BB, D), lambda i, f: (i, 0)),
            pl.BlockSpec((TK, D), lambda i, f: (0, 0)),
            pl.BlockSpec(memory_space=pl.ANY),
        ],
        out_specs=pl.BlockSpec(memory_space=pl.ANY),
        scratch_shapes=[
            pltpu.VMEM((B, D), jnp.float32),
            pltpu.SemaphoreType.DMA,
        ],
    )
    new_bank = pl.pallas_call(
        _body,
        grid_spec=grid_spec,
        out_shape=jax.ShapeDtypeStruct((T * M, D), jnp.float32),
        input_output_aliases={4: 0},
    )(flat_idx, task2d, features, flat_centers, bank_flat)

    return new_bank.reshape(T, M, D)


# speedup vs baseline: 1.0210x; 1.0210x over previous
import jax
import jax.numpy as jnp
from jax.experimental import pallas as pl
from jax.experimental.pallas import tpu as pltpu

B = 1024
D = 128
T = 4
K = 512
M = 65536
TK = T * K

_BB = 256
_STEPS = B // _BB


def _body(flat_sref, task_ref, feat_ref, cent_ref, bank_ref,
          out_ref, written, sem):
    del bank_ref
    i = pl.program_id(0)
    feats = feat_ref[...]
    cents = cent_ref[...]
    feats_bf = feats.astype(jnp.bfloat16)
    cents_bf = cents.astype(jnp.bfloat16)
    task = task_ref[...]
    ut = jax.lax.broadcasted_iota(jnp.int32, (_BB, 5), 1)
    u = jnp.where(ut == 0, 1.0,
                  jnp.where(ut - 1 == task, 0.0, 1.0)).astype(jnp.bfloat16)
    sq_col = jax.lax.dot_general(
        cents * cents, jnp.ones((D, 1), jnp.float32), (((1,), (0,)), ((), ())),
        preferred_element_type=jnp.float32)
    wt = jax.lax.broadcasted_iota(jnp.int32, (TK, 5), 1)
    wc = jax.lax.broadcasted_iota(jnp.int32, (TK, 5), 0)
    big = jnp.float32(1e9)
    w = jnp.where(wt == 0, sq_col,
                  jnp.where(wc // K == wt - 1, big, 0.0)).astype(jnp.bfloat16)
    fa = jnp.concatenate([feats_bf, u], axis=1)
    ca = jnp.concatenate([cents_bf * jnp.bfloat16(-2.0), w], axis=1)
    masked = jax.lax.dot_general(
        fa, ca, (((1,), (1,)), ((), ())),
        preferred_element_type=jnp.float32)
    mins = jnp.min(masked, axis=1, keepdims=True)
    onehot = (masked == mins).astype(jnp.bfloat16)
    sel = jax.lax.dot_general(
        onehot, cents_bf, (((1,), (0,)), ((), ())),
        preferred_element_type=jnp.float32)
    written[pl.ds(i * _BB, _BB), :] = feats + 0.1 * sel

    def issue(j, _):
        s = i * _BB + j
        pltpu.make_async_copy(
            written.at[pl.ds(s, 1), :],
            out_ref.at[pl.ds(flat_sref[s], 1), :],
            sem,
        ).start()
        return 0

    jax.lax.fori_loop(0, _BB, issue, 0, unroll=16)

    @pl.when(i == _STEPS - 1)
    def _drain():
        pltpu.make_async_copy(
            written.at[...],
            out_ref.at[pl.ds(0, B), :],
            sem,
        ).wait()


def kernel(features, feature_bank, cluster_centers, task_idx, write_idx):
    flat_centers = cluster_centers.reshape(TK, D)
    task2d = task_idx.reshape(B, 1)
    bank_flat = feature_bank.reshape(T * M, D)
    flat_idx = task_idx * M + write_idx

    grid_spec = pltpu.PrefetchScalarGridSpec(
        num_scalar_prefetch=1,
        grid=(_STEPS,),
        in_specs=[
            pl.BlockSpec((_BB, 1), lambda i, f: (i, 0)),
            pl.BlockSpec((_BB, D), lambda i, f: (i, 0)),
            pl.BlockSpec((TK, D), lambda i, f: (0, 0)),
            pl.BlockSpec(memory_space=pl.ANY),
        ],
        out_specs=pl.BlockSpec(memory_space=pl.ANY),
        scratch_shapes=[
            pltpu.VMEM((B, D), jnp.float32),
            pltpu.SemaphoreType.DMA,
        ],
    )
    new_bank = pl.pallas_call(
        _body,
        grid_spec=grid_spec,
        out_shape=jax.ShapeDtypeStruct((T * M, D), jnp.float32),
        input_output_aliases={4: 0},
    )(flat_idx, task2d, features, flat_centers, bank_flat)

    return new_bank.reshape(T, M, D)
